# baseline (speedup 1.0000x reference)
import functools

import jax
import jax.numpy as jnp
from jax import lax
from jax.experimental import pallas as pl
from jax.experimental.pallas import tpu as pltpu
from jax.experimental.pallas import tpu_sc as plsc

VOCAB = 1000000
D = 64
B = 4096
L = 200
NW = 32
BPW = B // NW
NGRP = BPW // 16


_K = 512
_F = _K // D
_R = VOCAB * D // _K
_TC_BLK = 1000


def _tc_dot_body(emb_ref, w_ref, out_ref):
    out_ref[...] = jnp.dot(
        emb_ref[...], w_ref[...], preferred_element_type=jnp.float32
    )


def _tc_dot(emb2, w2):
    return pl.pallas_call(
        _tc_dot_body,
        grid=(_R // _TC_BLK,),
        in_specs=[
            pl.BlockSpec((_TC_BLK, _K), lambda i: (i, 0)),
            pl.BlockSpec((_K, _F), lambda i: (0, 0)),
        ],
        out_specs=pl.BlockSpec((_TC_BLK, _F), lambda i: (i, 0)),
        out_shape=jax.ShapeDtypeStruct((_R, _F), jnp.float32),
    )(emb2, w2)


def _sc_body(a_hbm, v_hbm, bias_hbm, out_hbm, idx_v, vals_v, acc_v, bias_v, sem):
    wid = lax.axis_index("s") * 2 + lax.axis_index("c")
    pltpu.sync_copy(a_hbm.at[wid], idx_v)
    pltpu.sync_copy(bias_hbm, bias_v)

    def _fire(j, carry):
        pltpu.async_copy(v_hbm.at[idx_v.at[j]], vals_v.at[j], sem)
        return carry

    lax.fori_loop(0, L, _fire, 0)
    pltpu.make_async_copy(a_hbm.at[wid], idx_v, sem).wait()

    bias = bias_v[...]

    def _acc(j, accs):
        return tuple(
            accs[g] + vals_v[j, pl.ds(g * 16, 16)] for g in range(NGRP)
        )

    accs = lax.fori_loop(
        0, L, _acc, tuple(jnp.zeros((16,), jnp.float32) for _ in range(NGRP))
    )
    for g in range(NGRP):
        acc_v[pl.ds(g * 16, 16)] = accs[g] + bias
    pltpu.sync_copy(acc_v, out_hbm.at[pl.ds(wid * BPW, BPW)])


def _sc_gather_sum(a, v_flat, bias16):
    mesh = plsc.VectorSubcoreMesh(core_axis_name="c", subcore_axis_name="s")
    f = pl.kernel(
        _sc_body,
        mesh=mesh,
        out_type=jax.ShapeDtypeStruct((B,), jnp.float32),
        scratch_types=[
            pltpu.VMEM((L, BPW), jnp.int32),
            pltpu.VMEM((L, BPW), jnp.float32),
            pltpu.VMEM((BPW,), jnp.float32),
            pltpu.VMEM((16,), jnp.float32),
            pltpu.SemaphoreType.DMA,
        ],
    )
    return f(a, v_flat, bias16)


def kernel(inp, emb_table, fc_w, fc_b):
    w = (fc_w.astype(jnp.float32) / L).reshape(D)
    eye = jnp.eye(_F, dtype=jnp.float32)
    w2 = (eye[:, None, :] * w[None, :, None]).reshape(_K, _F)
    v = _tc_dot(emb_table.reshape(_R, _K), w2).reshape(VOCAB)
    a = inp.astype(jnp.int32).reshape(NW, BPW, L).transpose(0, 2, 1)
    bias16 = jnp.broadcast_to(fc_b.astype(jnp.float32), (16,))
    return _sc_gather_sum(a, v, bias16)


# speedup vs baseline: 1.0624x; 1.0624x over previous
import jax
import jax.numpy as jnp
from jax import lax
from jax.experimental import pallas as pl
from jax.experimental.pallas import tpu as pltpu
from jax.experimental.pallas import tpu_sc as plsc

VOCAB = 1000000
D = 64
B = 4096
L = 200
NW = 32
BPW = B // NW
NGRP = BPW // 16


_K = 512
_F = _K // D
_R = VOCAB * D // _K
_TC_BLK = 1024
_GRID = -(-_R // _TC_BLK)
_VPAD = _GRID * _TC_BLK


def _tc_dot_body(emb_ref, w_ref, out_ref):
    y = jnp.dot(emb_ref[...], w_ref[...], preferred_element_type=jnp.float32)
    out_ref[...] = y.T


def _tc_dot(emb2, w2):
    return pl.pallas_call(
        _tc_dot_body,
        grid=(_GRID,),
        in_specs=[
            pl.BlockSpec((_TC_BLK, _K), lambda i: (i, 0)),
            pl.BlockSpec((_K, _F), lambda i: (0, 0)),
        ],
        out_specs=pl.BlockSpec((_F, _TC_BLK), lambda i: (0, i)),
        out_shape=jax.ShapeDtypeStruct((_F, _VPAD), jnp.float32),
    )(emb2, w2)


def _sc_body(a_hbm, v_hbm, bias_hbm, out_hbm, idx_v, vals_v, acc_v, bias_v, sem):
    wid = lax.axis_index("s") * 2 + lax.axis_index("c")
    pltpu.sync_copy(a_hbm.at[wid], idx_v)
    pltpu.sync_copy(bias_hbm, bias_v)

    def _xform(j, carry):
        for g in range(NGRP):
            t = idx_v[j, pl.ds(g * 16, 16)]
            idx_v[j, pl.ds(g * 16, 16)] = (
                lax.shift_right_logical(t, 3) + (t & 7) * _VPAD
            )
        return carry

    lax.fori_loop(0, L, _xform, 0)

    def _fire(j, carry):
        pltpu.async_copy(v_hbm.at[idx_v.at[j]], vals_v.at[j], sem)
        return carry

    lax.fori_loop(0, L, _fire, 0)
    pltpu.make_async_copy(a_hbm.at[wid], idx_v, sem).wait()

    bias = bias_v[...]

    def _acc(j, accs):
        return tuple(
            accs[g] + vals_v[j, pl.ds(g * 16, 16)] for g in range(NGRP)
        )

    accs = lax.fori_loop(
        0, L, _acc, tuple(jnp.zeros((16,), jnp.float32) for _ in range(NGRP))
    )
    for g in range(NGRP):
        acc_v[pl.ds(g * 16, 16)] = accs[g] + bias
    pltpu.sync_copy(acc_v, out_hbm.at[pl.ds(wid * BPW, BPW)])


def _sc_gather_sum(a, v_flat, bias16):
    mesh = plsc.VectorSubcoreMesh(core_axis_name="c", subcore_axis_name="s")
    f = pl.kernel(
        _sc_body,
        mesh=mesh,
        out_type=jax.ShapeDtypeStruct((B,), jnp.float32),
        scratch_types=[
            pltpu.VMEM((L, BPW), jnp.int32),
            pltpu.VMEM((L, BPW), jnp.float32),
            pltpu.VMEM((BPW,), jnp.float32),
            pltpu.VMEM((16,), jnp.float32),
            pltpu.SemaphoreType.DMA,
        ],
    )
    return f(a, v_flat, bias16)


def kernel(inp, emb_table, fc_w, fc_b):
    w = (fc_w.astype(jnp.float32) / L).reshape(D)
    eye = jnp.eye(_F, dtype=jnp.float32)
    w2 = (eye[:, None, :] * w[None, :, None]).reshape(_K, _F)
    v2 = _tc_dot(emb_table.reshape(_R, _K), w2)
    a = inp.astype(jnp.int32).reshape(NW, BPW, L).transpose(0, 2, 1)
    bias16 = jnp.broadcast_to(fc_b.astype(jnp.float32), (16,))
    return _sc_gather_sum(a, v2.reshape(_F * _VPAD), bias16)


# speedup vs baseline: 1.5434x; 1.4528x over previous
import jax
import jax.numpy as jnp
from jax import lax
from jax.experimental import pallas as pl
from jax.experimental.pallas import tpu as pltpu
from jax.experimental.pallas import tpu_sc as plsc

VOCAB = 1000000
D = 64
B = 4096
L = 200
NW = 32
BPW = B // NW
NGRP = BPW // 16


_NS = 4
_BLK = 8192
_STEP = _NS * _BLK
_TCG = -(-VOCAB // _STEP)
_VPAD = _TCG * _STEP


def _tc_dot_body(*refs):
    w_ref, o_ref = refs[_NS], refs[_NS + 1]
    for s in range(_NS):
        y = jnp.dot(refs[s][...], w_ref[...], preferred_element_type=jnp.float32)
        o_ref[:, s * _BLK:(s + 1) * _BLK] = y.T


_TOTB = -(-VOCAB // _BLK)


def _mk_emb_map(s):
    return lambda i: (jnp.minimum(_NS * i + s, _TOTB - 1), 0)


def _tc_dot(emb_table, w2d):
    return pl.pallas_call(
        _tc_dot_body,
        grid=(_TCG,),
        in_specs=[pl.BlockSpec((_BLK, D), _mk_emb_map(s)) for s in range(_NS)]
        + [pl.BlockSpec((D, 1), lambda i: (0, 0))],
        out_specs=pl.BlockSpec((1, _STEP), lambda i: (0, i)),
        out_shape=jax.ShapeDtypeStruct((1, _VPAD), jnp.float32),
    )(*([emb_table] * _NS), w2d)


def _sc_body(a_hbm, v_hbm, bias_hbm, out_hbm, idx_v, vals_v, acc_v, bias_v, sem):
    wid = lax.axis_index("s") * 2 + lax.axis_index("c")
    pltpu.sync_copy(a_hbm.at[wid], idx_v)
    pltpu.sync_copy(bias_hbm, bias_v)

    def _fire(j, carry):
        pltpu.async_copy(v_hbm.at[idx_v.at[j]], vals_v.at[j], sem)
        return carry

    lax.fori_loop(0, L, _fire, 0)
    pltpu.make_async_copy(a_hbm.at[wid], idx_v, sem).wait()

    bias = bias_v[...]

    def _acc(j, accs):
        return tuple(
            accs[g] + vals_v[j, pl.ds(g * 16, 16)] for g in range(NGRP)
        )

    accs = lax.fori_loop(
        0, L, _acc, tuple(jnp.zeros((16,), jnp.float32) for _ in range(NGRP))
    )
    for g in range(NGRP):
        acc_v[pl.ds(g * 16, 16)] = accs[g] + bias
    pltpu.sync_copy(acc_v, out_hbm.at[pl.ds(wid * BPW, BPW)])


def _sc_gather_sum(a, v_flat, bias16):
    mesh = plsc.VectorSubcoreMesh(core_axis_name="c", subcore_axis_name="s")
    f = pl.kernel(
        _sc_body,
        mesh=mesh,
        out_type=jax.ShapeDtypeStruct((B,), jnp.float32),
        scratch_types=[
            pltpu.VMEM((L, BPW), jnp.int32),
            pltpu.VMEM((L, BPW), jnp.float32),
            pltpu.VMEM((BPW,), jnp.float32),
            pltpu.VMEM((16,), jnp.float32),
            pltpu.SemaphoreType.DMA,
        ],
    )
    return f(a, v_flat, bias16)


def kernel(inp, emb_table, fc_w, fc_b):
    w2d = fc_w.astype(jnp.float32) / L
    v2 = _tc_dot(emb_table, w2d)
    a = inp.astype(jnp.int32).reshape(NW, BPW, L).transpose(0, 2, 1)
    bias16 = jnp.broadcast_to(fc_b.astype(jnp.float32), (16,))
    return _sc_gather_sum(a, v2.reshape(_VPAD), bias16)
